# baseline (speedup 1.0000x reference)
import jax
import jax.numpy as jnp
from jax import lax
from jax.experimental import pallas as pl
from jax.experimental.pallas import tpu as pltpu
from jax.experimental.pallas import tpu_sc as plsc

B, L = 4096, 200
E = 64
H = E // 4
P = 1000000 + 1
NUM_HOURS = 24 + 1
HP = 32

NC, NS = 2, 16
NW = NC * NS

N_TOK = B * L
HB = B // 2

PROJ_BLK = 4096
N_PROJ_BLKS = -(-P // (2 * PROJ_BLK))
P_HALF = N_PROJ_BLKS * PROJ_BLK
TBL_MINOR_BLKS = -(-P // PROJ_BLK)

N_PIPE = 5
LC = L // N_PIPE
N_TOKC = N_TOK // N_PIPE
CHUNK = 512
OUT_ROWS_PER_W = (N_TOKC // 2) // NW
N_OCHUNKS = OUT_ROWS_PER_W // CHUNK


def _proj_body(pa_ref, pb_ref, w1_ref, htab_ref, w2_ref, b_ref,
               out_ref, hp_ref):
    dn = (((0,), (1,)), ((), ()))
    pa = lax.dot_general(pa_ref[...], w1_ref[...], dn,
                         preferred_element_type=jnp.float32)
    pb = lax.dot_general(pb_ref[...], w1_ref[...], dn,
                         preferred_element_type=jnp.float32)
    out_ref[...] = jnp.concatenate([pa, pb], axis=1)
    hp_ref[...] = lax.dot_general(
        htab_ref[...], w2_ref[...], (((1,), (1,)), ((), ())),
        preferred_element_type=jnp.float32) + b_ref[...]


def _tc_project(poi_t, w1, htab_pad, w2, fc_b):
    return pl.pallas_call(
        _proj_body,
        grid=(N_PROJ_BLKS,),
        in_specs=[
            pl.BlockSpec((E, PROJ_BLK), lambda i: (0, i)),
            pl.BlockSpec((E, PROJ_BLK),
                         lambda i: (0, jnp.minimum(i + N_PROJ_BLKS,
                                                   TBL_MINOR_BLKS - 1))),
            pl.BlockSpec((E, E), lambda i: (0, 0)),
            pl.BlockSpec((HP, H), lambda i: (0, 0)),
            pl.BlockSpec((E, H), lambda i: (0, 0)),
            pl.BlockSpec((1, E), lambda i: (0, 0)),
        ],
        out_specs=[
            pl.BlockSpec((PROJ_BLK, 2 * E), lambda i: (i, 0)),
            pl.BlockSpec((HP, E), lambda i: (0, 0)),
        ],
        out_shape=[
            jax.ShapeDtypeStruct((P_HALF, 2 * E), jnp.float32),
            jax.ShapeDtypeStruct((HP, E), jnp.float32),
        ],
        compiler_params=pltpu.CompilerParams(
            dimension_semantics=("parallel",)),
    )(poi_t, poi_t, w1, htab_pad, w2, fc_b)


def _sc_gather_body(idx_hbm, table_hbm, out_hbm,
                    idx_a, idx_b, rows_a, rows_b, sem):
    wid = lax.axis_index("s") * NC + lax.axis_index("c")
    base = wid * OUT_ROWS_PER_W

    def step(i, carry):
        r0 = base + i * CHUNK
        off_a = r0 + (r0 // HB) * HB
        off_b = off_a + HB
        pltpu.sync_copy(idx_hbm.at[pl.ds(off_a, CHUNK)], idx_a)
        pltpu.sync_copy(idx_hbm.at[pl.ds(off_b, CHUNK)], idx_b)
        pltpu.async_copy(table_hbm.at[idx_a], rows_a, sem).wait()
        pltpu.async_copy(table_hbm.at[idx_b], rows_b, sem).wait()
        pltpu.sync_copy(rows_a, out_hbm.at[pl.ds(r0, CHUNK), pl.ds(0, E)])
        pltpu.sync_copy(rows_b, out_hbm.at[pl.ds(r0, CHUNK), pl.ds(E, E)])
        return carry

    lax.fori_loop(0, N_OCHUNKS, step, 0)


def _sc_gather(token_flat, table_lin):
    mesh = plsc.VectorSubcoreMesh(
        core_axis_name="c", subcore_axis_name="s",
        num_cores=NC, num_subcores=NS,
    )
    kern = pl.kernel(
        _sc_gather_body,
        out_type=jax.ShapeDtypeStruct((N_TOKC // 2, 2 * E), jnp.float32),
        mesh=mesh,
        compiler_params=pltpu.CompilerParams(use_tc_tiling_on_sc=False),
        scratch_types=[
            pltpu.VMEM((CHUNK,), jnp.int32),
            pltpu.VMEM((CHUNK,), jnp.int32),
            pltpu.VMEM((CHUNK, E), jnp.float32),
            pltpu.VMEM((CHUNK, E), jnp.float32),
            pltpu.SemaphoreType.DMA,
        ],
    )
    return kern(token_flat, table_lin)


def _fin_body(g_ref, hid_ref, hp_ref, out_ref):
    g = g_ref[...]
    hid = hid_ref[0, 0, :]
    iota = lax.broadcasted_iota(jnp.int32, (HP, HB), 0)
    oha = (hid[None, :HB] == iota).astype(jnp.float32)
    ohb = (hid[None, HB:] == iota).astype(jnp.float32)
    hp = hp_ref[...]
    dn = (((0,), (0,)), ((), ()))
    hca = lax.dot_general(hp, oha, dn,
                          preferred_element_type=jnp.float32)
    hcb = lax.dot_general(hp, ohb, dn,
                          preferred_element_type=jnp.float32)
    ya = jnp.tanh(g[:, :E].T + hca)
    yb = jnp.tanh(g[:, E:].T + hcb)
    out_ref[0] = jnp.concatenate([ya, yb], axis=1)


def _tc_finish(gathered_k, hour_blk, hproj, k):
    return pl.pallas_call(
        _fin_body,
        grid=(LC,),
        in_specs=[
            pl.BlockSpec((HB, 2 * E), lambda i: (i, 0)),
            pl.BlockSpec((1, 1, B), lambda i: (i + k * LC, 0, 0)),
            pl.BlockSpec((HP, E), lambda i: (0, 0)),
        ],
        out_specs=pl.BlockSpec((1, E, B), lambda i: (i, 0, 0)),
        out_shape=jax.ShapeDtypeStruct((LC, E, B), jnp.float32),
        compiler_params=pltpu.CompilerParams(
            dimension_semantics=("parallel",)),
    )(gathered_k, hour_blk, hproj)


@jax.jit
def kernel(token_seq, hour_seq, poi_table, hour_table, fc_w, fc_b):
    htab_pad = jnp.zeros((HP, H), jnp.float32).at[:NUM_HOURS].set(hour_table)
    proj, hproj = _tc_project(poi_table.T, fc_w[:, :E], htab_pad, fc_w[:, E:],
                              fc_b.reshape(1, E))
    table_lin = proj.reshape(2 * P_HALF, E)

    tok = token_seq.T.reshape(N_TOK).astype(jnp.int32)
    tok2 = jnp.where(tok < P_HALF, 2 * tok, 2 * (tok - P_HALF) + 1)
    hour_blk = hour_seq.T.reshape(L, 1, B).astype(jnp.int32)

    outs = []
    for k in range(N_PIPE):
        idx_k = lax.slice(tok2, (k * N_TOKC,), ((k + 1) * N_TOKC,))
        gathered_k = _sc_gather(idx_k, table_lin)
        outs.append(_tc_finish(gathered_k, hour_blk, hproj, k))
    out = jnp.concatenate(outs, axis=0)
    return out.transpose(2, 0, 1)


# speedup vs baseline: 1.0484x; 1.0484x over previous
import jax
import jax.numpy as jnp
from jax import lax
from jax.experimental import pallas as pl
from jax.experimental.pallas import tpu as pltpu
from jax.experimental.pallas import tpu_sc as plsc

B, L = 4096, 200
E = 64
H = E // 4
P = 1000000 + 1
NUM_HOURS = 24 + 1
HP = 32

NC, NS = 2, 16
NW = NC * NS

N_TOK = B * L
HB = B // 2

PROJ_BLK = 4096
N_PROJ_BLKS = -(-P // (2 * PROJ_BLK))
P_HALF = N_PROJ_BLKS * PROJ_BLK
TBL_MINOR_BLKS = -(-P // PROJ_BLK)

N_PIPE = 5
LC = L // N_PIPE
N_TOKC = N_TOK // N_PIPE
CHUNK = 512
OUT_ROWS_PER_W = (N_TOKC // 2) // NW
N_OCHUNKS = OUT_ROWS_PER_W // CHUNK


def _proj_body(pa_ref, pb_ref, w1_ref, htab_ref, w2_ref, b_ref,
               out_ref, hp_ref):
    dn = (((0,), (1,)), ((), ()))
    pa = lax.dot_general(pa_ref[...], w1_ref[...], dn,
                         preferred_element_type=jnp.float32)
    pb = lax.dot_general(pb_ref[...], w1_ref[...], dn,
                         preferred_element_type=jnp.float32)
    out_ref[...] = jnp.concatenate([pa, pb], axis=1)
    hp_ref[...] = lax.dot_general(
        htab_ref[...], w2_ref[...], (((1,), (1,)), ((), ())),
        preferred_element_type=jnp.float32) + b_ref[...]


def _tc_project(poi_t, w1, htab_pad, w2, fc_b):
    return pl.pallas_call(
        _proj_body,
        grid=(N_PROJ_BLKS,),
        in_specs=[
            pl.BlockSpec((E, PROJ_BLK), lambda i: (0, i)),
            pl.BlockSpec((E, PROJ_BLK),
                         lambda i: (0, jnp.minimum(i + N_PROJ_BLKS,
                                                   TBL_MINOR_BLKS - 1))),
            pl.BlockSpec((E, E), lambda i: (0, 0)),
            pl.BlockSpec((HP, H), lambda i: (0, 0)),
            pl.BlockSpec((E, H), lambda i: (0, 0)),
            pl.BlockSpec((1, E), lambda i: (0, 0)),
        ],
        out_specs=[
            pl.BlockSpec((PROJ_BLK, 2 * E), lambda i: (i, 0)),
            pl.BlockSpec((HP, E), lambda i: (0, 0)),
        ],
        out_shape=[
            jax.ShapeDtypeStruct((P_HALF, 2 * E), jnp.float32),
            jax.ShapeDtypeStruct((HP, E), jnp.float32),
        ],
        compiler_params=pltpu.CompilerParams(
            dimension_semantics=("parallel",)),
    )(poi_t, poi_t, w1, htab_pad, w2, fc_b)


def _sc_gather_body(idx_hbm, table_hbm, out_hbm,
                    idx_a, idx_b, rows_a, rows_b, sem):
    wid = lax.axis_index("s") * NC + lax.axis_index("c")
    base = wid * OUT_ROWS_PER_W

    def step(i, carry):
        r0 = base + i * CHUNK
        off_a = r0 + (r0 // HB) * HB
        off_b = off_a + HB
        pltpu.sync_copy(idx_hbm.at[pl.ds(off_a, CHUNK)], idx_a)
        pltpu.sync_copy(idx_hbm.at[pl.ds(off_b, CHUNK)], idx_b)
        pltpu.async_copy(table_hbm.at[idx_a], rows_a, sem).wait()
        pltpu.async_copy(table_hbm.at[idx_b], rows_b, sem).wait()
        pltpu.sync_copy(rows_a, out_hbm.at[pl.ds(r0, CHUNK), pl.ds(0, E)])
        pltpu.sync_copy(rows_b, out_hbm.at[pl.ds(r0, CHUNK), pl.ds(E, E)])
        return carry

    lax.fori_loop(0, N_OCHUNKS, step, 0)


def _sc_gather(token_flat, table_lin):
    mesh = plsc.VectorSubcoreMesh(
        core_axis_name="c", subcore_axis_name="s",
        num_cores=NC, num_subcores=NS,
    )
    kern = pl.kernel(
        _sc_gather_body,
        out_type=jax.ShapeDtypeStruct((N_TOKC // 2, 2 * E), jnp.float32),
        mesh=mesh,
        compiler_params=pltpu.CompilerParams(use_tc_tiling_on_sc=False),
        scratch_types=[
            pltpu.VMEM((CHUNK,), jnp.int32),
            pltpu.VMEM((CHUNK,), jnp.int32),
            pltpu.VMEM((CHUNK, E), jnp.float32),
            pltpu.VMEM((CHUNK, E), jnp.float32),
            pltpu.SemaphoreType.DMA,
        ],
    )
    return kern(token_flat, table_lin)


def _fin_body(g_ref, hid_ref, hp_ref, out_ref):
    g = g_ref[...]
    hid = hid_ref[0, 0, :]
    iota = lax.broadcasted_iota(jnp.int32, (HP, HB), 0)
    oha = (hid[None, :HB] == iota).astype(jnp.float32)
    ohb = (hid[None, HB:] == iota).astype(jnp.float32)
    hp = hp_ref[...]
    dn = (((0,), (0,)), ((), ()))
    hca = lax.dot_general(hp, oha, dn,
                          preferred_element_type=jnp.float32)
    hcb = lax.dot_general(hp, ohb, dn,
                          preferred_element_type=jnp.float32)
    ya = jnp.tanh(g[:, :E].T + hca)
    yb = jnp.tanh(g[:, E:].T + hcb)
    out_ref[0] = jnp.concatenate([ya, yb], axis=1)


def _fin_body_acc(acc_ref, g_ref, hid_ref, hp_ref, out_ref):
    _fin_body(g_ref, hid_ref, hp_ref, out_ref)


def _tc_finish(gathered_k, hour_blk, hproj, k, acc):
    return pl.pallas_call(
        _fin_body_acc,
        grid=(LC,),
        in_specs=[
            pl.BlockSpec((1, 8, 128), lambda i: (0, 0, 0)),
            pl.BlockSpec((HB, 2 * E), lambda i: (i, 0)),
            pl.BlockSpec((1, 1, B), lambda i: (i + k * LC, 0, 0)),
            pl.BlockSpec((HP, E), lambda i: (0, 0)),
        ],
        out_specs=pl.BlockSpec((1, E, B), lambda i: (i + k * LC, 0, 0)),
        out_shape=jax.ShapeDtypeStruct((L, E, B), jnp.float32),
        input_output_aliases={0: 0},
        compiler_params=pltpu.CompilerParams(
            dimension_semantics=("parallel",)),
    )(acc, gathered_k, hour_blk, hproj)


@jax.jit
def kernel(token_seq, hour_seq, poi_table, hour_table, fc_w, fc_b):
    htab_pad = jnp.zeros((HP, H), jnp.float32).at[:NUM_HOURS].set(hour_table)
    proj, hproj = _tc_project(poi_table.T, fc_w[:, :E], htab_pad, fc_w[:, E:],
                              fc_b.reshape(1, E))
    table_lin = proj.reshape(2 * P_HALF, E)

    tok = token_seq.T.reshape(N_TOK).astype(jnp.int32)
    tok2 = jnp.where(tok < P_HALF, 2 * tok, 2 * (tok - P_HALF) + 1)
    hour_blk = hour_seq.T.reshape(L, 1, B).astype(jnp.int32)

    gathered = [
        _sc_gather(lax.slice(tok2, (k * N_TOKC,), ((k + 1) * N_TOKC,)),
                   table_lin)
        for k in range(N_PIPE)
    ]
    acc = jnp.empty((L, E, B), jnp.float32)
    for k in range(N_PIPE):
        acc = _tc_finish(gathered[k], hour_blk, hproj, k, acc)
    return acc.transpose(2, 0, 1)


# speedup vs baseline: 1.2170x; 1.1609x over previous
import jax
import jax.numpy as jnp
from jax import lax
from jax.experimental import pallas as pl
from jax.experimental.pallas import tpu as pltpu
from jax.experimental.pallas import tpu_sc as plsc

B, L = 4096, 200
E = 64
H = E // 4
P = 1000000 + 1
NUM_HOURS = 24 + 1
HP = 32

NC, NS = 2, 16
NW = NC * NS

N_TOK = B * L
HB = B // 2

PROJ_BLK = 4096
N_PROJ_BLKS = -(-P // (2 * PROJ_BLK))
P_HALF = N_PROJ_BLKS * PROJ_BLK
TBL_MINOR_BLKS = -(-P // PROJ_BLK)

N_PIPE = 5
LC = L // N_PIPE
N_TOKC = N_TOK // N_PIPE
CHUNK = 512
OUT_ROWS_PER_W = (N_TOKC // 2) // NW
N_OCHUNKS = OUT_ROWS_PER_W // CHUNK


def _proj_body(pa_ref, pb_ref, w1_ref, htab_ref, w2_ref, b_ref,
               out_ref, hp_ref):
    dn = (((0,), (1,)), ((), ()))
    pa = lax.dot_general(pa_ref[...], w1_ref[...], dn,
                         preferred_element_type=jnp.float32)
    pb = lax.dot_general(pb_ref[...], w1_ref[...], dn,
                         preferred_element_type=jnp.float32)
    out_ref[...] = jnp.concatenate([pa, pb], axis=1)
    hp_ref[...] = lax.dot_general(
        htab_ref[...], w2_ref[...], (((1,), (1,)), ((), ())),
        preferred_element_type=jnp.float32) + b_ref[...]


def _tc_project(poi_t, w1, htab_pad, w2, fc_b):
    return pl.pallas_call(
        _proj_body,
        grid=(N_PROJ_BLKS,),
        in_specs=[
            pl.BlockSpec((E, PROJ_BLK), lambda i: (0, i)),
            pl.BlockSpec((E, PROJ_BLK),
                         lambda i: (0, jnp.minimum(i + N_PROJ_BLKS,
                                                   TBL_MINOR_BLKS - 1))),
            pl.BlockSpec((E, E), lambda i: (0, 0)),
            pl.BlockSpec((HP, H), lambda i: (0, 0)),
            pl.BlockSpec((E, H), lambda i: (0, 0)),
            pl.BlockSpec((1, E), lambda i: (0, 0)),
        ],
        out_specs=[
            pl.BlockSpec((PROJ_BLK, 2 * E), lambda i: (i, 0)),
            pl.BlockSpec((HP, E), lambda i: (0, 0)),
        ],
        out_shape=[
            jax.ShapeDtypeStruct((P_HALF, 2 * E), jnp.float32),
            jax.ShapeDtypeStruct((HP, E), jnp.float32),
        ],
        compiler_params=pltpu.CompilerParams(
            dimension_semantics=("parallel",)),
    )(poi_t, poi_t, w1, htab_pad, w2, fc_b)


def _sc_gather_body(idx_hbm, table_hbm, out_hbm,
                    idx_a, idx_b, rows_a, rows_b, sem):
    wid = lax.axis_index("s") * NC + lax.axis_index("c")
    base = wid * OUT_ROWS_PER_W

    def step(i, carry):
        r0 = base + i * CHUNK
        off_a = r0 + (r0 // HB) * HB
        off_b = off_a + HB
        pltpu.sync_copy(idx_hbm.at[pl.ds(off_a, CHUNK)], idx_a)
        pltpu.sync_copy(idx_hbm.at[pl.ds(off_b, CHUNK)], idx_b)
        pltpu.async_copy(table_hbm.at[idx_a], rows_a, sem).wait()
        pltpu.async_copy(table_hbm.at[idx_b], rows_b, sem).wait()
        pltpu.sync_copy(rows_a, out_hbm.at[pl.ds(r0, CHUNK), pl.ds(0, E)])
        pltpu.sync_copy(rows_b, out_hbm.at[pl.ds(r0, CHUNK), pl.ds(E, E)])
        return carry

    lax.fori_loop(0, N_OCHUNKS, step, 0)


def _sc_gather(token_flat, table_lin):
    mesh = plsc.VectorSubcoreMesh(
        core_axis_name="c", subcore_axis_name="s",
        num_cores=NC, num_subcores=NS,
    )
    kern = pl.kernel(
        _sc_gather_body,
        out_type=jax.ShapeDtypeStruct((N_TOKC // 2, 2 * E), jnp.float32),
        mesh=mesh,
        compiler_params=pltpu.CompilerParams(use_tc_tiling_on_sc=False),
        scratch_types=[
            pltpu.VMEM((CHUNK,), jnp.int32),
            pltpu.VMEM((CHUNK,), jnp.int32),
            pltpu.VMEM((CHUNK, E), jnp.float32),
            pltpu.VMEM((CHUNK, E), jnp.float32),
            pltpu.SemaphoreType.DMA,
        ],
    )
    return kern(token_flat, table_lin)


def _fin_body(g_ref, hid_ref, hp_ref, out_ref):
    g = g_ref[...]
    hid = hid_ref[0, 0, :]
    iota = lax.broadcasted_iota(jnp.int32, (HP, HB), 0)
    oha = (hid[None, :HB] == iota).astype(jnp.float32)
    ohb = (hid[None, HB:] == iota).astype(jnp.float32)
    hp = hp_ref[...]
    dn = (((0,), (0,)), ((), ()))
    hca = lax.dot_general(hp, oha, dn,
                          preferred_element_type=jnp.float32)
    hcb = lax.dot_general(hp, ohb, dn,
                          preferred_element_type=jnp.float32)
    ya = jnp.tanh(g[:, :E].T + hca)
    yb = jnp.tanh(g[:, E:].T + hcb)
    out_ref[0] = jnp.concatenate([ya, yb], axis=1)


def _fin_body_acc(acc_ref, g_ref, hid_ref, hp_ref, out_ref):
    _fin_body(g_ref, hid_ref, hp_ref, out_ref)


def _tc_finish(gathered_k, hour_blk, hproj, k, acc):
    data_specs = [
        pl.BlockSpec((HB, 2 * E), lambda i: (i, 0)),
        pl.BlockSpec((1, 1, B), lambda i: (i + k * LC, 0, 0)),
        pl.BlockSpec((HP, E), lambda i: (0, 0)),
    ]
    common = dict(
        grid=(LC,),
        out_specs=pl.BlockSpec((1, E, B), lambda i: (i + k * LC, 0, 0)),
        out_shape=jax.ShapeDtypeStruct((L, E, B), jnp.float32),
        compiler_params=pltpu.CompilerParams(
            dimension_semantics=("parallel",)),
    )
    if k == 0:
        return pl.pallas_call(
            _fin_body, in_specs=data_specs, **common,
        )(gathered_k, hour_blk, hproj)
    return pl.pallas_call(
        _fin_body_acc,
        in_specs=[pl.BlockSpec((1, 8, 128), lambda i: (0, 0, 0))]
        + data_specs,
        input_output_aliases={0: 0},
        **common,
    )(acc, gathered_k, hour_blk, hproj)


@jax.jit
def kernel(token_seq, hour_seq, poi_table, hour_table, fc_w, fc_b):
    htab_pad = jnp.zeros((HP, H), jnp.float32).at[:NUM_HOURS].set(hour_table)
    proj, hproj = _tc_project(poi_table.T, fc_w[:, :E], htab_pad, fc_w[:, E:],
                              fc_b.reshape(1, E))
    table_lin = proj.reshape(2 * P_HALF, E)

    tok = token_seq.T.reshape(N_TOK).astype(jnp.int32)
    tok2 = jnp.where(tok < P_HALF, 2 * tok, 2 * (tok - P_HALF) + 1)
    hour_blk = hour_seq.T.reshape(L, 1, B).astype(jnp.int32)

    gathered = [
        _sc_gather(lax.slice(tok2, (k * N_TOKC,), ((k + 1) * N_TOKC,)),
                   table_lin)
        for k in range(N_PIPE)
    ]
    acc = None
    for k in range(N_PIPE):
        acc = _tc_finish(gathered[k], hour_blk, hproj, k, acc)
    return acc.transpose(2, 0, 1)
